# Initial kernel scaffold; baseline (speedup 1.0000x reference)
import jax
import jax.numpy as jnp
from jax.experimental import pallas as pl


def kernel(action_id, embedding_table):
    raise NotImplementedError("write your pallas kernel here")


# speedup vs baseline: 5.3026x; 5.3026x over previous
import functools

import jax
import jax.numpy as jnp
from jax import lax
from jax.experimental import pallas as pl
from jax.experimental.pallas import tpu as pltpu
from jax.experimental.pallas import tpu_sc as plsc

NC = 2
NS = 16
NW = NC * NS
D = 64
SEG = 128
K = 5
RG = SEG * K


def _make_kernel(B):
    b_per_w = B // NW
    n_seg = b_per_w // SEG
    n_grp = b_per_w // RG
    mesh = plsc.VectorSubcoreMesh(core_axis_name="c", subcore_axis_name="s")

    @functools.partial(
        pl.kernel,
        mesh=mesh,
        out_type=jax.ShapeDtypeStruct((B, D), jnp.float32),
        scratch_types=[
            pltpu.VMEM((n_seg, SEG), jnp.int32),
            pltpu.VMEM((RG, D), jnp.float32),
            pltpu.SemaphoreType.DMA,
        ],
        compiler_params=pltpu.CompilerParams(use_tc_tiling_on_sc=False),
    )
    def k(idx_hbm, table_hbm, out_hbm, idx_v, rows_v, sem):
        wid = lax.axis_index("s") * NC + lax.axis_index("c")
        pltpu.sync_copy(idx_hbm.at[wid], idx_v)
        base = wid * b_per_w

        def body(g, carry):
            cps = [
                pltpu.async_copy(
                    table_hbm.at[idx_v.at[g * K + j]],
                    rows_v.at[pl.ds(j * SEG, SEG)],
                    sem,
                )
                for j in range(K)
            ]
            for c in cps:
                c.wait()
            pltpu.sync_copy(rows_v, out_hbm.at[pl.ds(base + g * RG, RG)])
            return carry

        lax.fori_loop(0, n_grp, body, 0)

    return k


def kernel(action_id, embedding_table):
    Bq, T = action_id.shape
    B = Bq * T
    idx = action_id.reshape(NW, B // NW // SEG, SEG).astype(jnp.int32)
    out = _make_kernel(B)(idx, embedding_table)
    return out.reshape(Bq, T, D)


# speedup vs baseline: 7.3006x; 1.3768x over previous
import functools

import jax
import jax.numpy as jnp
from jax import lax
from jax.experimental import pallas as pl
from jax.experimental.pallas import tpu as pltpu
from jax.experimental.pallas import tpu_sc as plsc

NC = 2
NS = 16
NW = NC * NS
D = 64
V = 1000
SEG = 128
K = 2
RG = SEG * K


def _make_kernel(B):
    b_per_w = B // NW
    n_seg = b_per_w // SEG
    n_grp = b_per_w // RG
    mesh = plsc.VectorSubcoreMesh(core_axis_name="c", subcore_axis_name="s")

    @functools.partial(
        pl.kernel,
        mesh=mesh,
        out_type=jax.ShapeDtypeStruct((B, D), jnp.float32),
        scratch_types=[
            pltpu.VMEM_SHARED((V, D), jnp.float32),
            pltpu.VMEM((n_seg, SEG), jnp.int32),
            pltpu.VMEM((2, RG, D), jnp.float32),
            pltpu.SemaphoreType.DMA,
            pltpu.SemaphoreType.DMA,
        ],
        compiler_params=pltpu.CompilerParams(use_tc_tiling_on_sc=False),
    )
    def k(idx_hbm, table_hbm, out_hbm, table_v, idx_v, rows_v, sem0, sem1):
        wid = lax.axis_index("s") * NC + lax.axis_index("c")
        base = wid * b_per_w
        sems = [sem0, sem1]

        @pl.when(lax.axis_index("s") == 0)
        def _stage_table():
            pltpu.sync_copy(table_hbm, table_v)

        pltpu.sync_copy(idx_hbm.at[wid], idx_v)
        plsc.subcore_barrier()

        def fire(g, b):
            for j in range(K):
                pltpu.async_copy(
                    table_v.at[idx_v.at[g * K + j]],
                    rows_v.at[b, pl.ds(j * SEG, SEG)],
                    sems[b],
                )

        def drain(b):
            pltpu.make_async_copy(
                out_hbm.at[pl.ds(0, RG)], rows_v.at[b], sems[b]
            ).wait()

        def write(g, b):
            pltpu.sync_copy(rows_v.at[b], out_hbm.at[pl.ds(base + g * RG, RG)])

        fire(0, 0)

        def body(i, carry):
            g0 = 2 * i
            fire(g0 + 1, 1)
            drain(0)
            write(g0, 0)
            fire(g0 + 2, 0)
            drain(1)
            write(g0 + 1, 1)
            return carry

        lax.fori_loop(0, n_grp // 2 - 1, body, 0)
        g0 = n_grp - 2
        fire(g0 + 1, 1)
        drain(0)
        write(g0, 0)
        drain(1)
        write(g0 + 1, 1)

    return k


def kernel(action_id, embedding_table):
    Bq, T = action_id.shape
    B = Bq * T
    idx = action_id.reshape(NW, B // NW // SEG, SEG).astype(jnp.int32)
    out = _make_kernel(B)(idx, embedding_table)
    return out.reshape(Bq, T, D)
